# baseline (speedup 1.0000x reference)
import jax
import jax.numpy as jnp
from jax import lax
from jax.experimental import pallas as pl
from jax.experimental.pallas import tpu as pltpu


def _linear_kernel(x_ref, w_ref, b_ref, o_ref):
    w_bf = w_ref[...].astype(jnp.bfloat16)
    m = x_ref.shape[0]
    bm = m // 4
    for mo in (0, bm, 2 * bm, 3 * bm):
        acc = lax.dot_general(
            x_ref[pl.ds(mo, bm), :], w_bf,
            dimension_numbers=(((1,), (1,)), ((), ())),
            preferred_element_type=jnp.float32,
        )
        o_ref[pl.ds(mo, bm), :] = acc + b_ref[...]


def _pick_bn(n):
    for bn in (256, 128):
        if n % bn == 0:
            return bn
    return n


def kernel(x, W, b):
    M, K = x.shape
    N = W.shape[0]
    BN = _pick_bn(N)

    x_bf = x.astype(jnp.bfloat16)
    b2 = b.reshape(1, N)

    out = pl.pallas_call(
        _linear_kernel,
        grid=(N // BN,),
        in_specs=[
            pl.BlockSpec((M, K), lambda i: (0, 0)),
            pl.BlockSpec((BN, K), lambda i: (i, 0)),
            pl.BlockSpec((1, BN), lambda i: (0, i)),
        ],
        out_specs=pl.BlockSpec((M, BN), lambda i: (0, i)),
        out_shape=jax.ShapeDtypeStruct((M, N), jnp.float32),
        compiler_params=pltpu.CompilerParams(
            dimension_semantics=("arbitrary",),
        ),
    )(x_bf, W, b2)
    return out


# speedup vs baseline: 1.0063x; 1.0063x over previous
import jax
import jax.numpy as jnp
from jax import lax
from jax.experimental import pallas as pl
from jax.experimental.pallas import tpu as pltpu

_CONV = 16


def _linear_kernel(x_ref, w_ref, b_ref, o_ref, xbf_ref):
    i = pl.program_id(0)
    m, ck = x_ref.shape
    bn = w_ref.shape[0]

    @pl.when(i < _CONV)
    def _convert():
        xbf_ref[:, pl.ds(i * ck, ck)] = x_ref[...].astype(jnp.bfloat16)

    @pl.when(i >= _CONV)
    def _matmul():
        w_bf = w_ref[...].astype(jnp.bfloat16)
        bm = m // 2
        for mo in (0, bm):
            acc = lax.dot_general(
                xbf_ref[pl.ds(mo, bm), :], w_bf,
                dimension_numbers=(((1,), (1,)), ((), ())),
                preferred_element_type=jnp.float32,
            )
            o_ref[pl.ds(mo, bm), :] = acc + b_ref[...]


def _pick_bn(n):
    for bn in (256, 128):
        if n % bn == 0:
            return bn
    return n


def kernel(x, W, b):
    M, K = x.shape
    N = W.shape[0]
    BN = _pick_bn(N)
    CK = K // _CONV

    b2 = b.reshape(1, N)

    out = pl.pallas_call(
        _linear_kernel,
        grid=(_CONV + N // BN,),
        in_specs=[
            pl.BlockSpec((M, CK), lambda i: (0, jnp.minimum(i, _CONV - 1))),
            pl.BlockSpec((BN, K), lambda i: (jnp.maximum(i - _CONV, 0), 0)),
            pl.BlockSpec((1, BN), lambda i: (0, jnp.maximum(i - _CONV, 0))),
        ],
        out_specs=pl.BlockSpec((M, BN), lambda i: (0, jnp.maximum(i - _CONV, 0))),
        out_shape=jax.ShapeDtypeStruct((M, N), jnp.float32),
        scratch_shapes=[pltpu.VMEM((M, K), jnp.bfloat16)],
        compiler_params=pltpu.CompilerParams(
            dimension_semantics=("arbitrary",),
        ),
    )(x, W, b2)
    return out


# speedup vs baseline: 1.0090x; 1.0027x over previous
import jax
import jax.numpy as jnp
from jax import lax
from jax.experimental import pallas as pl
from jax.experimental.pallas import tpu as pltpu

_CONV = 16


def _linear_kernel(x_ref, w_ref, b_ref, o_ref, xbf_ref):
    i = pl.program_id(0)
    m, ck = x_ref.shape
    bn = w_ref.shape[0]

    @pl.when(i < _CONV)
    def _convert():
        xbf_ref[:, pl.ds(i * ck, ck)] = x_ref[...].astype(jnp.bfloat16)

    @pl.when(i >= _CONV)
    def _matmul():
        w_bf = w_ref[...].astype(jnp.bfloat16)
        bm = m // 2
        kk = w_ref.shape[1]
        hk = kk // 2
        for mo in (0, bm):
            acc = lax.dot_general(
                xbf_ref[pl.ds(mo, bm), pl.ds(0, hk)], w_bf[:, :hk],
                dimension_numbers=(((1,), (1,)), ((), ())),
                preferred_element_type=jnp.float32,
            )
            acc = acc + lax.dot_general(
                xbf_ref[pl.ds(mo, bm), pl.ds(hk, hk)], w_bf[:, hk:],
                dimension_numbers=(((1,), (1,)), ((), ())),
                preferred_element_type=jnp.float32,
            )
            o_ref[pl.ds(mo, bm), :] = acc + b_ref[...]


def _pick_bn(n):
    for bn in (256, 128):
        if n % bn == 0:
            return bn
    return n


def kernel(x, W, b):
    M, K = x.shape
    N = W.shape[0]
    BN = _pick_bn(N)
    CK = K // _CONV

    b2 = b.reshape(1, N)

    out = pl.pallas_call(
        _linear_kernel,
        grid=(_CONV + N // BN,),
        in_specs=[
            pl.BlockSpec((M, CK), lambda i: (0, jnp.minimum(i, _CONV - 1))),
            pl.BlockSpec((BN, K), lambda i: (jnp.maximum(i - _CONV, 0), 0)),
            pl.BlockSpec((1, BN), lambda i: (0, jnp.maximum(i - _CONV, 0))),
        ],
        out_specs=pl.BlockSpec((M, BN), lambda i: (0, jnp.maximum(i - _CONV, 0))),
        out_shape=jax.ShapeDtypeStruct((M, N), jnp.float32),
        scratch_shapes=[pltpu.VMEM((M, K), jnp.bfloat16)],
        compiler_params=pltpu.CompilerParams(
            dimension_semantics=("arbitrary",),
        ),
    )(x, W, b2)
    return out
